# Initial kernel scaffold; baseline (speedup 1.0000x reference)
import jax
import jax.numpy as jnp
from jax.experimental import pallas as pl


def kernel(inp, table):
    raise NotImplementedError("write your pallas kernel here")


# speedup vs baseline: 1.0941x; 1.0941x over previous
import functools

import jax
import jax.numpy as jnp
from jax import lax
from jax.experimental import pallas as pl
from jax.experimental.pallas import tpu as pltpu
from jax.experimental.pallas import tpu_sc as plsc

NUM_EMB = 1000001
D = 32
BATCH = 16384
HIST = 50
TOT = BATCH * HIST

NUM_CORES = 2
NUM_SUBCORES = 16
NW = NUM_CORES * NUM_SUBCORES
PER_W = TOT // NW
CHUNK = 1024
STEPS = PER_W // CHUNK

_mesh = plsc.VectorSubcoreMesh(core_axis_name="c", subcore_axis_name="s")


@functools.partial(
    pl.kernel,
    mesh=_mesh,
    out_type=jax.ShapeDtypeStruct((TOT, D), jnp.float32),
    compiler_params=pltpu.CompilerParams(use_tc_tiling_on_sc=False),
    scratch_types=[
        pltpu.VMEM((CHUNK,), jnp.int32),
        pltpu.VMEM((CHUNK, D), jnp.float32),
        pltpu.SemaphoreType.DMA,
    ],
)
def _gather_kernel(idx_hbm, table_hbm, out_hbm, idx_v, rows_v, sem):
    wid = lax.axis_index("s") * NUM_CORES + lax.axis_index("c")
    base = wid * PER_W

    def body(i, carry):
        off = base + i * CHUNK
        pltpu.sync_copy(idx_hbm.at[pl.ds(off, CHUNK)], idx_v)
        pltpu.async_copy(table_hbm.at[idx_v], rows_v, sem).wait()
        pltpu.sync_copy(rows_v, out_hbm.at[pl.ds(off, CHUNK)])
        return carry

    lax.fori_loop(0, STEPS, body, 0)


def kernel(inp, table):
    flat = jnp.asarray(inp, jnp.int32).reshape(TOT)
    out = _gather_kernel(flat, table)
    return out.reshape(BATCH, HIST, D)


# speedup vs baseline: 1.1067x; 1.0115x over previous
import functools

import jax
import jax.numpy as jnp
from jax import lax
from jax.experimental import pallas as pl
from jax.experimental.pallas import tpu as pltpu
from jax.experimental.pallas import tpu_sc as plsc

NUM_EMB = 1000001
D = 32
BATCH = 16384
HIST = 50
TOT = BATCH * HIST

NUM_CORES = 2
NUM_SUBCORES = 16
NW = NUM_CORES * NUM_SUBCORES
PER_W = TOT // NW

K = 4
CHUNK = 400
STEPS = PER_W // CHUNK
ROUNDS = STEPS // K

_mesh = plsc.VectorSubcoreMesh(core_axis_name="c", subcore_axis_name="s")

_scratch = [pltpu.VMEM((PER_W,), jnp.int32)]
_scratch += [pltpu.VMEM((CHUNK, D), jnp.float32) for _ in range(2 * K)]
_scratch += [pltpu.SemaphoreType.DMA for _ in range(4 * K)]


@functools.partial(
    pl.kernel,
    mesh=_mesh,
    out_type=jax.ShapeDtypeStruct((TOT, D), jnp.float32),
    compiler_params=pltpu.CompilerParams(use_tc_tiling_on_sc=False),
    scratch_types=_scratch,
)
def _gather_kernel(idx_hbm, table_hbm, out_hbm, idx_v, *bufs_and_sems):
    rows = [bufs_and_sems[g * K + b] for g in range(2) for b in range(K)]
    rows = [rows[:K], rows[K:]]
    sems = bufs_and_sems[2 * K:]
    gsem = [sems[:K], sems[K:2 * K]]
    ssem = [sems[2 * K:3 * K], sems[3 * K:]]

    wid = lax.axis_index("s") * NUM_CORES + lax.axis_index("c")
    base = wid * PER_W

    pltpu.sync_copy(idx_hbm.at[pl.ds(base, PER_W)], idx_v)

    def gather_desc(r, g, b):
        off = (r * K + b) * CHUNK
        return pltpu.make_async_copy(
            table_hbm.at[idx_v.at[pl.ds(off, CHUNK)]], rows[g][b], gsem[g][b])

    def store_desc(r, g, b):
        off = base + (r * K + b) * CHUNK
        return pltpu.make_async_copy(
            rows[g][b], out_hbm.at[pl.ds(off, CHUNK)], ssem[g][b])

    def fire_gathers(r, g):
        for b in range(K):
            gather_desc(r, g, b).start()

    def wait_gathers(r, g):
        for b in range(K):
            gather_desc(r, g, b).wait()

    def fire_stores(r, g):
        for b in range(K):
            store_desc(r, g, b).start()

    def wait_stores(r, g):
        for b in range(K):
            store_desc(r, g, b).wait()

    fire_gathers(0, 0)
    wait_gathers(0, 0)
    fire_stores(0, 0)
    fire_gathers(1, 1)

    def pair_body(t, carry):
        r = 1 + 2 * t
        wait_stores(r - 1, 0)
        wait_gathers(r, 1)
        fire_stores(r, 1)
        fire_gathers(r + 1, 0)
        r2 = r + 1
        wait_stores(r2 - 1, 1)
        wait_gathers(r2, 0)
        fire_stores(r2, 0)
        fire_gathers(r2 + 1, 1)
        return carry

    lax.fori_loop(0, (ROUNDS - 2) // 2, pair_body, 0)

    rl = ROUNDS - 1
    wait_stores(rl - 1, 0)
    wait_gathers(rl, 1)
    fire_stores(rl, 1)
    wait_stores(rl, 1)


def kernel(inp, table):
    flat = jnp.asarray(inp, jnp.int32).reshape(TOT)
    out = _gather_kernel(flat, table)
    return out.reshape(BATCH, HIST, D)


# speedup vs baseline: 1.7915x; 1.6188x over previous
import functools

import jax
import jax.numpy as jnp
from jax import lax
from jax.experimental import pallas as pl
from jax.experimental.pallas import tpu as pltpu
from jax.experimental.pallas import tpu_sc as plsc

NUM_EMB = 1000001
D = 32
BATCH = 16384
HIST = 50
TOT = BATCH * HIST

NUM_CORES = 2
NUM_SUBCORES = 16
NW = NUM_CORES * NUM_SUBCORES
PER_W = TOT // NW

K = 4
CHUNK = 400
STEPS = PER_W // CHUNK
ROUNDS = STEPS // K

_mesh = plsc.VectorSubcoreMesh(core_axis_name="c", subcore_axis_name="s")

ROWS_PER_CHUNK = CHUNK // HIST

_scratch = [pltpu.VMEM((PER_W,), jnp.int32)]
_scratch += [pltpu.VMEM((CHUNK, D), jnp.float32) for _ in range(2 * K)]
_scratch += [pltpu.SemaphoreType.DMA for _ in range(4 * K)]


@functools.partial(
    pl.kernel,
    mesh=_mesh,
    out_type=jax.ShapeDtypeStruct((BATCH, HIST, D), jnp.float32),
    compiler_params=pltpu.CompilerParams(use_tc_tiling_on_sc=False),
    scratch_types=_scratch,
)
def _gather_kernel(idx_hbm, table_hbm, out_3d, idx_v, *bufs_and_sems):
    rows = [bufs_and_sems[g * K + b] for g in range(2) for b in range(K)]
    rows = [rows[:K], rows[K:]]
    sems = bufs_and_sems[2 * K:]
    gsem = [sems[:K], sems[K:2 * K]]
    ssem = [sems[2 * K:3 * K], sems[3 * K:]]

    wid = lax.axis_index("s") * NUM_CORES + lax.axis_index("c")
    base = wid * PER_W

    pltpu.sync_copy(idx_hbm.at[pl.ds(base, PER_W)], idx_v)

    def gather_desc(r, g, b):
        off = (r * K + b) * CHUNK
        return pltpu.make_async_copy(
            table_hbm.at[idx_v.at[pl.ds(off, CHUNK)]], rows[g][b], gsem[g][b])

    def store_descs(r, g, b):
        brow = (base + (r * K + b) * CHUNK) // HIST
        return [
            pltpu.make_async_copy(
                rows[g][b].at[pl.ds(rr * HIST, HIST)],
                out_3d.at[brow + rr], ssem[g][b])
            for rr in range(ROWS_PER_CHUNK)
        ]

    def fire_gathers(r, g):
        for b in range(K):
            gather_desc(r, g, b).start()

    def wait_gathers(r, g):
        for b in range(K):
            gather_desc(r, g, b).wait()

    def fire_stores(r, g):
        for b in range(K):
            for d in store_descs(r, g, b):
                d.start()

    def wait_stores(r, g):
        for b in range(K):
            for d in store_descs(r, g, b):
                d.wait()

    fire_gathers(0, 0)
    wait_gathers(0, 0)
    fire_stores(0, 0)
    fire_gathers(1, 1)

    def pair_body(t, carry):
        r = 1 + 2 * t
        wait_stores(r - 1, 0)
        wait_gathers(r, 1)
        fire_stores(r, 1)
        fire_gathers(r + 1, 0)
        r2 = r + 1
        wait_stores(r2 - 1, 1)
        wait_gathers(r2, 0)
        fire_stores(r2, 0)
        fire_gathers(r2 + 1, 1)
        return carry

    lax.fori_loop(0, (ROUNDS - 2) // 2, pair_body, 0)

    rl = ROUNDS - 1
    wait_stores(rl - 1, 0)
    wait_gathers(rl, 1)
    fire_stores(rl, 1)
    wait_stores(rl, 1)


def kernel(inp, table):
    flat = jnp.asarray(inp, jnp.int32).reshape(TOT)
    return _gather_kernel(flat, table)
